# Initial kernel scaffold; baseline (speedup 1.0000x reference)
import jax
import jax.numpy as jnp
from jax.experimental import pallas as pl


def kernel(x, router_logits, w13, w2):
    raise NotImplementedError("write your pallas kernel here")


# speedup vs baseline: 1.0709x; 1.0709x over previous
import functools
import jax
import jax.numpy as jnp
from jax.experimental import pallas as pl
from jax.experimental.pallas import tpu as pltpu

E = 8
K = 2
H = 1024
I = 2048
T = 2048
TK = T * K

BM = 256
BN = 512
NB = 23
NPAD = NB * BM
NI = I // BN


def _gmm_body(meta_ref, x_ref, w13g_ref, w13u_ref, w2_ref, ws_ref, o_ref):
    b = pl.program_id(0)
    n = pl.program_id(1)
    nb_total = meta_ref[NB]

    @pl.when(b < nb_total)
    def _():
        xb = x_ref[...]
        g = jax.lax.dot_general(xb, w13g_ref[0], (((1,), (1,)), ((), ())),
                                preferred_element_type=jnp.float32)
        u = jax.lax.dot_general(xb, w13u_ref[0], (((1,), (1,)), ((), ())),
                                preferred_element_type=jnp.float32)
        act = g * jax.lax.logistic(g) * u
        py = jax.lax.dot_general(act, w2_ref[0], (((1,), (1,)), ((), ())),
                                 preferred_element_type=jnp.float32)
        py = py * ws_ref[...]

        @pl.when(n == 0)
        def _():
            o_ref[...] = py

        @pl.when(n > 0)
        def _():
            o_ref[...] += py


@jax.jit
def _gmm(meta, x_sorted, w13, w2, w_sorted):
    grid_spec = pltpu.PrefetchScalarGridSpec(
        num_scalar_prefetch=1,
        grid=(NB, NI),
        in_specs=[
            pl.BlockSpec((BM, H), lambda b, n, m: (b, 0)),
            pl.BlockSpec((1, BN, H), lambda b, n, m: (m[b], n, 0)),
            pl.BlockSpec((1, BN, H), lambda b, n, m: (m[b], n + NI, 0)),
            pl.BlockSpec((1, H, BN), lambda b, n, m: (m[b], 0, n)),
            pl.BlockSpec((BM, 1), lambda b, n, m: (b, 0)),
        ],
        out_specs=pl.BlockSpec((BM, H), lambda b, n, m: (b, 0)),
    )
    return pl.pallas_call(
        _gmm_body,
        grid_spec=grid_spec,
        out_shape=jax.ShapeDtypeStruct((NPAD, H), jnp.float32),
    )(meta, x_sorted, w13, w13, w2, w_sorted)


def kernel(x, router_logits, w13, w2):
    probs = jax.nn.softmax(router_logits.astype(jnp.float32), axis=-1)
    topk_vals, topk_ids = jax.lax.top_k(probs, K)
    topk_vals = topk_vals / jnp.sum(topk_vals, axis=-1, keepdims=True)

    e_flat = topk_ids.reshape(TK).astype(jnp.int32)
    w_flat = topk_vals.reshape(TK).astype(jnp.float32)
    tok = jnp.arange(TK, dtype=jnp.int32) // K

    onehot = (e_flat[:, None] == jnp.arange(E, dtype=jnp.int32)[None, :]).astype(jnp.int32)
    cums = jnp.cumsum(onehot, axis=0)
    rank = jnp.take_along_axis(cums, e_flat[:, None], axis=1)[:, 0] - 1
    counts = cums[-1]
    nblk = (counts + BM - 1) // BM
    blk_off = jnp.concatenate([jnp.zeros(1, jnp.int32),
                               jnp.cumsum(nblk).astype(jnp.int32)])
    total_blocks = blk_off[E]
    pad_off = blk_off[:E] * BM
    pos = pad_off[e_flat] + rank

    expert_of = jnp.clip(
        (jnp.arange(NB, dtype=jnp.int32)[:, None] >= blk_off[None, 1:]).sum(
            axis=1, dtype=jnp.int32), 0, E - 1)
    meta = jnp.concatenate([expert_of, total_blocks[None]]).astype(jnp.int32)

    x_sorted = jnp.zeros((NPAD, H), jnp.float32).at[pos].set(x[tok])
    w_sorted = jnp.zeros((NPAD, 1), jnp.float32).at[pos, 0].set(w_flat)

    y = _gmm(meta, x_sorted, w13, w2, w_sorted)
    out = y[pos[0::K]] + y[pos[1::K]]
    return out


# speedup vs baseline: 1.2850x; 1.2000x over previous
import functools
import jax
import jax.numpy as jnp
from jax import lax
from jax.experimental import pallas as pl
from jax.experimental.pallas import tpu as pltpu
from jax.experimental.pallas import tpu_sc as plsc

E = 8
K = 2
H = 1024
I = 2048
T = 2048
TK = T * K

BM = 256
BN = 512
NB = 23
NPAD = NB * BM
NI = I // BN


NC = 2
NS = 16
NW = NC * NS
PW = TK // NW
CH = 32
NCH = PW // CH

_SC_MESH = plsc.VectorSubcoreMesh(core_axis_name="c", subcore_axis_name="s")


@functools.partial(
    pl.kernel,
    mesh=_SC_MESH,
    out_type=jax.ShapeDtypeStruct((NPAD, H), jnp.float32),
    scratch_types=[
        pltpu.VMEM((NCH, CH), jnp.int32),
        pltpu.VMEM((NCH, CH), jnp.int32),
        pltpu.VMEM((CH, H), jnp.float32),
        pltpu.SemaphoreType.DMA,
        pltpu.SemaphoreType.DMA,
    ],
)
def _dispatch(x_hbm, pos_hbm, tok_hbm, xs_hbm, pos_v, tok_v, rows_v, sem_g, sem_s):
    wid = lax.axis_index("s") * NC + lax.axis_index("c")
    pltpu.sync_copy(pos_hbm.at[wid], pos_v)
    pltpu.sync_copy(tok_hbm.at[wid], tok_v)
    for j in range(NCH):
        pltpu.async_copy(x_hbm.at[tok_v.at[j]], rows_v, sem_g).wait()
        pltpu.async_copy(rows_v, xs_hbm.at[pos_v.at[j]], sem_s).wait()


def _gmm_body(meta_ref, x_ref, w13g_ref, w13u_ref, w2_ref, ws_ref, o_ref):
    b = pl.program_id(0)
    n = pl.program_id(1)
    nb_total = meta_ref[NB]

    @pl.when(b < nb_total)
    def _():
        xb = x_ref[...]
        g = jax.lax.dot_general(xb, w13g_ref[0], (((1,), (1,)), ((), ())),
                                preferred_element_type=jnp.float32)
        u = jax.lax.dot_general(xb, w13u_ref[0], (((1,), (1,)), ((), ())),
                                preferred_element_type=jnp.float32)
        act = g * jax.lax.logistic(g) * u
        py = jax.lax.dot_general(act, w2_ref[0], (((1,), (1,)), ((), ())),
                                 preferred_element_type=jnp.float32)
        py = py * ws_ref[...]

        @pl.when(n == 0)
        def _():
            o_ref[...] = py

        @pl.when(n > 0)
        def _():
            o_ref[...] += py


@jax.jit
def _gmm(meta, x_sorted, w13, w2, w_sorted):
    grid_spec = pltpu.PrefetchScalarGridSpec(
        num_scalar_prefetch=1,
        grid=(NB, NI),
        in_specs=[
            pl.BlockSpec((BM, H), lambda b, n, m: (b, 0)),
            pl.BlockSpec((1, BN, H), lambda b, n, m: (m[b], n, 0)),
            pl.BlockSpec((1, BN, H), lambda b, n, m: (m[b], n + NI, 0)),
            pl.BlockSpec((1, H, BN), lambda b, n, m: (m[b], 0, n)),
            pl.BlockSpec((BM, 1), lambda b, n, m: (b, 0)),
        ],
        out_specs=pl.BlockSpec((BM, H), lambda b, n, m: (b, 0)),
    )
    return pl.pallas_call(
        _gmm_body,
        grid_spec=grid_spec,
        out_shape=jax.ShapeDtypeStruct((NPAD, H), jnp.float32),
    )(meta, x_sorted, w13, w13, w2, w_sorted)


def kernel(x, router_logits, w13, w2):
    probs = jax.nn.softmax(router_logits.astype(jnp.float32), axis=-1)
    topk_vals, topk_ids = jax.lax.top_k(probs, K)
    topk_vals = topk_vals / jnp.sum(topk_vals, axis=-1, keepdims=True)

    e_flat = topk_ids.reshape(TK).astype(jnp.int32)
    w_flat = topk_vals.reshape(TK).astype(jnp.float32)
    tok = jnp.arange(TK, dtype=jnp.int32) // K

    onehot = (e_flat[:, None] == jnp.arange(E, dtype=jnp.int32)[None, :]).astype(jnp.int32)
    cums = jnp.cumsum(onehot, axis=0)
    rank = jnp.take_along_axis(cums, e_flat[:, None], axis=1)[:, 0] - 1
    counts = cums[-1]
    nblk = (counts + BM - 1) // BM
    blk_off = jnp.concatenate([jnp.zeros(1, jnp.int32),
                               jnp.cumsum(nblk).astype(jnp.int32)])
    total_blocks = blk_off[E]
    pad_off = blk_off[:E] * BM
    pos = pad_off[e_flat] + rank

    expert_of = jnp.clip(
        (jnp.arange(NB, dtype=jnp.int32)[:, None] >= blk_off[None, 1:]).sum(
            axis=1, dtype=jnp.int32), 0, E - 1)
    meta = jnp.concatenate([expert_of, total_blocks[None]]).astype(jnp.int32)

    pos3d = pos.reshape(NW, NCH, CH).astype(jnp.int32)
    tok3d = tok.reshape(NW, NCH, CH)
    x_sorted = _dispatch(x, pos3d, tok3d)
    w_sorted = jnp.zeros((NPAD, 1), jnp.float32).at[pos, 0].set(w_flat)

    y = _gmm(meta, x_sorted, w13, w2, w_sorted)
    out = y[pos[0::K]] + y[pos[1::K]]
    return out
